# baseline (speedup 1.0000x reference)
import functools

import jax
import jax.numpy as jnp
from jax import lax
from jax.experimental import pallas as pl
from jax.experimental.pallas import tpu as pltpu
from jax.experimental.pallas import tpu_sc as plsc

EMB_DIM = 32

_info = plsc.get_sparse_core_info()
_NC, _NS = _info.num_cores, _info.num_subcores
_NW = _NC * _NS

_K = 16


@functools.cache
def _make_gather(batch: int, fields: int):
    rows_per_w = batch // _NW
    assert batch % _NW == 0 and rows_per_w % _K == 0

    mesh = plsc.VectorSubcoreMesh(core_axis_name="c", subcore_axis_name="s")

    @functools.partial(
        pl.kernel,
        mesh=mesh,
        out_type=jax.ShapeDtypeStruct((batch, fields, EMB_DIM), jnp.float32),
        scratch_types=[
            pltpu.VMEM((rows_per_w, fields), jnp.int32),
            pltpu.VMEM((_K, fields, EMB_DIM), jnp.float32),
            pltpu.SemaphoreType.DMA((_K,)),
        ],
        compiler_params=pltpu.CompilerParams(use_tc_tiling_on_sc=False),
    )
    def gather_kernel(keys_hbm, table_hbm, out_hbm, idx_v, rows_v, gsem):
        wid = lax.axis_index("s") * _NC + lax.axis_index("c")
        base = wid * rows_per_w
        pltpu.sync_copy(keys_hbm.at[pl.ds(base, rows_per_w)], idx_v)

        def group(g, carry):
            copies = []
            for b in range(_K):
                r = g * _K + b
                copies.append(
                    pltpu.async_copy(
                        table_hbm.at[idx_v.at[r]],
                        rows_v.at[b],
                        gsem.at[b],
                    )
                )
            for b in range(_K):
                r = g * _K + b
                copies[b].wait()
                pltpu.sync_copy(rows_v.at[b], out_hbm.at[base + r])
            return carry

        lax.fori_loop(0, rows_per_w // _K, group, 0)

    return gather_kernel


def kernel(keys, table):
    batch, fields = keys.shape
    return _make_gather(batch, fields)(keys.astype(jnp.int32), table)
